# baseline (device time: 12768 ns/iter reference)
import jax
import jax.numpy as jnp
from jax import lax
from jax.experimental import pallas as pl
from jax.experimental.pallas import tpu as pltpu

T = 256
D = 512
V_LOCAL = 4096


def kernel(x, W, labels):
    def body(x_ref, w_ref, lab_ref, out_ref, pkt_ref, rbuf_ref,
             send_sem, recv_sem):
        my_x = lax.axis_index("x")
        my_y = lax.axis_index("y")
        my_z = lax.axis_index("z")
        peer = (1 - my_x, my_y, my_z)

        barrier = pltpu.get_barrier_semaphore()
        pl.semaphore_signal(
            barrier, inc=1, device_id=peer,
            device_id_type=pl.DeviceIdType.MESH,
        )
        pl.semaphore_wait(barrier, 1)

        xv = x_ref[...].astype(jnp.bfloat16)
        wv = w_ref[...].astype(jnp.bfloat16)
        logits = jnp.dot(xv, wv, preferred_element_type=jnp.float32)

        m = jnp.max(logits, axis=1)
        s = jnp.sum(jnp.exp(logits - m[:, None]), axis=1)

        col = lax.broadcasted_iota(jnp.int32, (T, V_LOCAL), 1)
        lab = lab_ref[...]
        t = jnp.sum(jnp.where(col == (lab - my_x * V_LOCAL), logits, 0.0),
                    axis=1)

        pkt_ref[0, :] = m
        pkt_ref[1, :] = s
        pkt_ref[2, :] = t

        rdma = pltpu.make_async_remote_copy(
            src_ref=pkt_ref,
            dst_ref=rbuf_ref,
            send_sem=send_sem,
            recv_sem=recv_sem,
            device_id=peer,
            device_id_type=pl.DeviceIdType.MESH,
        )
        rdma.start()
        rdma.wait()

        m2 = rbuf_ref[0, :]
        s2 = rbuf_ref[1, :]
        t2 = rbuf_ref[2, :]
        mm = jnp.maximum(m, m2)
        ss = s * jnp.exp(m - mm) + s2 * jnp.exp(m2 - mm)
        out_ref[...] = mm + jnp.log(ss) - (t + t2)

    return pl.pallas_call(
        body,
        out_shape=jax.ShapeDtypeStruct((T,), jnp.float32),
        in_specs=[
            pl.BlockSpec(memory_space=pltpu.VMEM),
            pl.BlockSpec(memory_space=pltpu.VMEM),
            pl.BlockSpec(memory_space=pltpu.VMEM),
        ],
        out_specs=pl.BlockSpec(memory_space=pltpu.VMEM),
        scratch_shapes=[
            pltpu.VMEM((3, T), jnp.float32),
            pltpu.VMEM((3, T), jnp.float32),
            pltpu.SemaphoreType.DMA,
            pltpu.SemaphoreType.DMA,
        ],
        compiler_params=pltpu.CompilerParams(collective_id=0),
    )(x, W, labels.reshape(T, 1))
